# baseline (speedup 1.0000x reference)
import dataclasses
import functools

import jax
import jax.numpy as jnp
from jax import lax
from jax.experimental import pallas as pl
from jax.experimental.pallas import tpu as pltpu
from jax.experimental.pallas import tpu_sc as plsc

_H = 384
_W = 384
_N = _H * _W
_BC = 384
_NC = 2
_NS = 16
_NW = _NC * _NS
_PER_W = _N // _NW
_G = 16
_NSTEP = _PER_W // _G
_L = 16
_GT = 128
_CPT = _GT // _G


def _sc_interp(table, idx4, wt4):
    mesh = plsc.VectorSubcoreMesh(core_axis_name="c", subcore_axis_name="s")
    cp = pltpu.CompilerParams()
    if "needs_layout_passes" in pltpu.CompilerParams.__dataclass_fields__:
        cp = dataclasses.replace(cp, needs_layout_passes=False)

    @functools.partial(
        pl.kernel,
        mesh=mesh,
        compiler_params=cp,
        out_type=jax.ShapeDtypeStruct((_N, _BC), jnp.float32),
        scratch_types=(
            [pltpu.VMEM((4 * _PER_W,), jnp.int32)]
            + [pltpu.VMEM((_PER_W,), jnp.float32) for _ in range(4)]
            + [
                pltpu.VMEM((2, 4 * _G, _BC), jnp.float32),
                pltpu.VMEM((2, _G, _BC), jnp.float32),
                pltpu.SemaphoreType.DMA,
                pltpu.SemaphoreType.DMA,
                pltpu.SemaphoreType.DMA,
                pltpu.SemaphoreType.DMA,
            ]
        ),
    )
    def k(table_h, idx_h, wt_h, out_h,
          idx_v, w0_v, w1_v, w2_v, w3_v, g, outb,
          gsem0, gsem1, osem0, osem1):
        w_vs = [w0_v, w1_v, w2_v, w3_v]
        gsems = [gsem0, gsem1]
        osems = [osem0, osem1]
        wid = lax.axis_index("s") * _NC + lax.axis_index("c")
        base = wid * _PER_W

        pltpu.sync_copy(idx_h.at[pl.ds(4 * base, 4 * _PER_W)], idx_v)
        for kk in range(4):
            pltpu.sync_copy(wt_h.at[kk, pl.ds(base, _PER_W)], w_vs[kk])

        def fire_gathers(chunk, par):
            off = chunk * 4 * _G
            pltpu.async_copy(table_h.at[idx_v.at[pl.ds(off, 4 * _G)]],
                             g.at[par], gsems[par])

        def drain_gathers(chunk, par):
            off = chunk * 4 * _G
            pltpu.make_async_copy(table_h.at[idx_v.at[pl.ds(off, 4 * _G)]],
                                  g.at[par], gsems[par]).wait()

        def fire_out(chunk, par):
            off = chunk * _G
            pltpu.async_copy(outb.at[par], out_h.at[pl.ds(base + off, _G)],
                             osems[par])

        def drain_out(par):
            pltpu.make_async_copy(outb.at[par], out_h.at[pl.ds(base, _G)],
                                  osems[par]).wait()

        def compute(chunk, par):
            off = chunk * _G

            @pl.loop(0, _G)
            def _(r):
                ridx = jnp.full((_L,), off + r, jnp.int32)
                ws = [plsc.load_gather(w_vs[kk], [ridx]) for kk in range(4)]
                for j in range(_BC // _L):
                    s = pl.ds(j * _L, _L)
                    acc = ws[0] * g[par, r, s]
                    acc = acc + ws[1] * g[par, _G + r, s]
                    acc = acc + ws[2] * g[par, 2 * _G + r, s]
                    acc = acc + ws[3] * g[par, 3 * _G + r, s]
                    outb[par, r, s] = acc

        fire_gathers(0, 0)

        @pl.loop(0, _NSTEP, step=2)
        def _(c0):
            for par in range(2):
                chunk = c0 + par

                @pl.when(chunk + 1 < _NSTEP)
                def _():
                    fire_gathers(chunk + 1, 1 - par)

                drain_gathers(chunk, par)

                @pl.when(chunk >= 2)
                def _():
                    drain_out(par)

                compute(chunk, par)
                fire_out(chunk, par)

        drain_out(0)
        drain_out(1)

    return k(table, idx4, wt4)


def _tc_transpose_wide(src, blk_c):
    rows, cols = src.shape

    def body(x_ref, o_ref):
        o_ref[...] = x_ref[...].T

    return pl.pallas_call(
        body,
        grid=(cols // blk_c,),
        in_specs=[pl.BlockSpec((rows, blk_c), lambda i: (0, i))],
        out_specs=pl.BlockSpec((blk_c, rows), lambda i: (i, 0)),
        out_shape=jax.ShapeDtypeStruct((cols, rows), src.dtype),
    )(src)


def _tc_transpose_tall(src, blk_r):
    rows, cols = src.shape

    def body(x_ref, o_ref):
        o_ref[...] = x_ref[...].T

    return pl.pallas_call(
        body,
        grid=(rows // blk_r,),
        in_specs=[pl.BlockSpec((blk_r, cols), lambda i: (i, 0))],
        out_specs=pl.BlockSpec((cols, blk_r), lambda i: (0, i)),
        out_shape=jax.ShapeDtypeStruct((cols, rows), src.dtype),
    )(src)


def _indices_weights(theta):
    cy = (_H - 1) / 2.0
    cx = (_W - 1) / 2.0
    gy, gx = jnp.meshgrid(
        jnp.arange(_H, dtype=jnp.float32) - cy,
        jnp.arange(_W, dtype=jnp.float32) - cx,
        indexing="ij",
    )
    cos_t = jnp.cos(theta)
    sin_t = jnp.sin(theta)
    src_x = cos_t * gx + sin_t * gy + cx
    src_y = -sin_t * gx + cos_t * gy + cy
    x0 = jnp.floor(src_x)
    y0 = jnp.floor(src_y)
    wx1 = src_x - x0
    wx0 = 1.0 - wx1
    wy1 = src_y - y0
    wy0 = 1.0 - wy1
    valid = ((src_x >= 0) & (src_x <= _W - 1)
             & (src_y >= 0) & (src_y <= _H - 1)).astype(jnp.float32)
    x0i = jnp.clip(x0, 0, _W - 1).astype(jnp.int32)
    x1i = jnp.clip(x0 + 1.0, 0, _W - 1).astype(jnp.int32)
    y0i = jnp.clip(y0, 0, _H - 1).astype(jnp.int32)
    y1i = jnp.clip(y0 + 1.0, 0, _H - 1).astype(jnp.int32)
    idx4 = jnp.stack([
        (y0i * _W + x0i).reshape(-1),
        (y0i * _W + x1i).reshape(-1),
        (y1i * _W + x0i).reshape(-1),
        (y1i * _W + x1i).reshape(-1),
    ])
    wt4 = jnp.stack([
        (wy0 * wx0 * valid).reshape(-1),
        (wy0 * wx1 * valid).reshape(-1),
        (wy1 * wx0 * valid).reshape(-1),
        (wy1 * wx1 * valid).reshape(-1),
    ])
    return idx4, wt4


def kernel(x, const):
    theta = jnp.squeeze(const, axis=0)[0]
    idx4, wt4 = _indices_weights(theta)
    idxc = (idx4.reshape(4, _NW, _NSTEP, _G)
            .transpose(1, 2, 0, 3).reshape(-1))
    table = _tc_transpose_wide(x.reshape(_BC, _N), 512)
    out_t = _sc_interp(table, idxc, wt4)
    return _tc_transpose_tall(out_t, 512).reshape(x.shape)


# speedup vs baseline: 1.1572x; 1.1572x over previous
import dataclasses
import functools

import jax
import jax.numpy as jnp
from jax import lax
from jax.experimental import pallas as pl
from jax.experimental.pallas import tpu as pltpu
from jax.experimental.pallas import tpu_sc as plsc

_H = 384
_W = 384
_N = _H * _W
_BC = 384
_NC = 2
_NS = 16
_NW = _NC * _NS
_PER_W = _N // _NW
_G = 16
_NSTEP = _PER_W // _G
_L = 16
_GT = 128
_CPT = _GT // _G


def _sc_interp(table, idx4, wt4):
    mesh = plsc.VectorSubcoreMesh(core_axis_name="c", subcore_axis_name="s")
    cp = pltpu.CompilerParams()
    if "needs_layout_passes" in pltpu.CompilerParams.__dataclass_fields__:
        cp = dataclasses.replace(cp, needs_layout_passes=False)

    @functools.partial(
        pl.kernel,
        mesh=mesh,
        compiler_params=cp,
        out_type=jax.ShapeDtypeStruct((_N, _BC), jnp.float32),
        scratch_types=(
            [pltpu.VMEM((4 * _PER_W,), jnp.int32)]
            + [pltpu.VMEM((_PER_W,), jnp.float32) for _ in range(4)]
            + [
                pltpu.VMEM((3, 4 * _G, _BC), jnp.float32),
                pltpu.VMEM((3, _G, _BC), jnp.float32),
                pltpu.SemaphoreType.DMA,
                pltpu.SemaphoreType.DMA,
                pltpu.SemaphoreType.DMA,
                pltpu.SemaphoreType.DMA,
                pltpu.SemaphoreType.DMA,
                pltpu.SemaphoreType.DMA,
            ]
        ),
    )
    def k(table_h, idx_h, wt_h, out_h,
          idx_v, w0_v, w1_v, w2_v, w3_v, g, outb,
          gsem0, gsem1, gsem2, osem0, osem1, osem2):
        w_vs = [w0_v, w1_v, w2_v, w3_v]
        gsems = [gsem0, gsem1, gsem2]
        osems = [osem0, osem1, osem2]
        wid = lax.axis_index("s") * _NC + lax.axis_index("c")
        base = wid * _PER_W

        pltpu.sync_copy(idx_h.at[pl.ds(4 * base, 4 * _PER_W)], idx_v)
        for kk in range(4):
            pltpu.sync_copy(wt_h.at[kk, pl.ds(base, _PER_W)], w_vs[kk])

        def fire_gathers(chunk, par):
            off = chunk * 4 * _G
            pltpu.async_copy(table_h.at[idx_v.at[pl.ds(off, 4 * _G)]],
                             g.at[par], gsems[par])

        def drain_gathers(chunk, par):
            off = chunk * 4 * _G
            pltpu.make_async_copy(table_h.at[idx_v.at[pl.ds(off, 4 * _G)]],
                                  g.at[par], gsems[par]).wait()

        def fire_out(chunk, par):
            off = chunk * _G
            pltpu.async_copy(outb.at[par], out_h.at[pl.ds(base + off, _G)],
                             osems[par])

        def drain_out(par):
            pltpu.make_async_copy(outb.at[par], out_h.at[pl.ds(base, _G)],
                                  osems[par]).wait()

        def compute(chunk, par):
            off = chunk * _G

            @pl.loop(0, _G)
            def _(r):
                ridx = jnp.full((_L,), off + r, jnp.int32)
                ws = [plsc.load_gather(w_vs[kk], [ridx]) for kk in range(4)]
                for j in range(_BC // _L):
                    s = pl.ds(j * _L, _L)
                    acc = ws[0] * g[par, r, s]
                    acc = acc + ws[1] * g[par, _G + r, s]
                    acc = acc + ws[2] * g[par, 2 * _G + r, s]
                    acc = acc + ws[3] * g[par, 3 * _G + r, s]
                    outb[par, r, s] = acc

        fire_gathers(0, 0)
        fire_gathers(1, 1)

        @pl.loop(0, _NSTEP, step=3)
        def _(c0):
            for par in range(3):
                chunk = c0 + par
                nxt_par = (par + 2) % 3

                @pl.when(chunk + 2 < _NSTEP)
                def _():
                    fire_gathers(chunk + 2, nxt_par)

                drain_gathers(chunk, par)

                @pl.when(chunk >= 3)
                def _():
                    drain_out(par)

                compute(chunk, par)
                fire_out(chunk, par)

        drain_out(0)
        drain_out(1)
        drain_out(2)

    return k(table, idx4, wt4)


def _tc_transpose_wide(src, blk_c):
    rows, cols = src.shape

    def body(x_ref, o_ref):
        o_ref[...] = x_ref[...].T

    return pl.pallas_call(
        body,
        grid=(cols // blk_c,),
        in_specs=[pl.BlockSpec((rows, blk_c), lambda i: (0, i))],
        out_specs=pl.BlockSpec((blk_c, rows), lambda i: (i, 0)),
        out_shape=jax.ShapeDtypeStruct((cols, rows), src.dtype),
    )(src)


def _tc_transpose_tall(src, blk_r):
    rows, cols = src.shape

    def body(x_ref, o_ref):
        o_ref[...] = x_ref[...].T

    return pl.pallas_call(
        body,
        grid=(rows // blk_r,),
        in_specs=[pl.BlockSpec((blk_r, cols), lambda i: (i, 0))],
        out_specs=pl.BlockSpec((cols, blk_r), lambda i: (0, i)),
        out_shape=jax.ShapeDtypeStruct((cols, rows), src.dtype),
    )(src)


def _indices_weights(theta):
    cy = (_H - 1) / 2.0
    cx = (_W - 1) / 2.0
    gy, gx = jnp.meshgrid(
        jnp.arange(_H, dtype=jnp.float32) - cy,
        jnp.arange(_W, dtype=jnp.float32) - cx,
        indexing="ij",
    )
    cos_t = jnp.cos(theta)
    sin_t = jnp.sin(theta)
    src_x = cos_t * gx + sin_t * gy + cx
    src_y = -sin_t * gx + cos_t * gy + cy
    x0 = jnp.floor(src_x)
    y0 = jnp.floor(src_y)
    wx1 = src_x - x0
    wx0 = 1.0 - wx1
    wy1 = src_y - y0
    wy0 = 1.0 - wy1
    valid = ((src_x >= 0) & (src_x <= _W - 1)
             & (src_y >= 0) & (src_y <= _H - 1)).astype(jnp.float32)
    x0i = jnp.clip(x0, 0, _W - 1).astype(jnp.int32)
    x1i = jnp.clip(x0 + 1.0, 0, _W - 1).astype(jnp.int32)
    y0i = jnp.clip(y0, 0, _H - 1).astype(jnp.int32)
    y1i = jnp.clip(y0 + 1.0, 0, _H - 1).astype(jnp.int32)
    idx4 = jnp.stack([
        (y0i * _W + x0i).reshape(-1),
        (y0i * _W + x1i).reshape(-1),
        (y1i * _W + x0i).reshape(-1),
        (y1i * _W + x1i).reshape(-1),
    ])
    wt4 = jnp.stack([
        (wy0 * wx0 * valid).reshape(-1),
        (wy0 * wx1 * valid).reshape(-1),
        (wy1 * wx0 * valid).reshape(-1),
        (wy1 * wx1 * valid).reshape(-1),
    ])
    return idx4, wt4


def kernel(x, const):
    theta = jnp.squeeze(const, axis=0)[0]
    idx4, wt4 = _indices_weights(theta)
    idxc = (idx4.reshape(4, _NW, _NSTEP, _G)
            .transpose(1, 2, 0, 3).reshape(-1))
    table = x.reshape(_BC, _N).T
    out_t = _sc_interp(table, idxc, wt4)
    return out_t.T.reshape(x.shape)
